# Initial kernel scaffold; baseline (speedup 1.0000x reference)
import jax
import jax.numpy as jnp
from jax.experimental import pallas as pl


def kernel(x, edge_index, edge_weight, Wz, bz, Wr, br, Wh, bh, Wl, bl):
    raise NotImplementedError("write your pallas kernel here")


# speedup vs baseline: 6.5135x; 6.5135x over previous
import dataclasses
import functools

import jax
import jax.numpy as jnp
from jax import lax
from jax.experimental import pallas as pl
from jax.experimental.pallas import tpu as pltpu
from jax.experimental.pallas import tpu_sc as plsc

N = 10000
E = 320000
F = 128
NS = 16
B = 64
MEGA = 512
EPT = 20480
E_PAD = EPT * NS
RPT = 624
REM = N - NS * RPT
EPS = 1e-30


def _sc_compiler_params():
  cp = pltpu.CompilerParams()
  if "needs_layout_passes" in pltpu.CompilerParams.__dataclass_fields__:
    cp = dataclasses.replace(cp, needs_layout_passes=False)
  return cp


def _deg_body(niA_hbm, w_hbm, deg_hbm,
              msg_v, niB_v, w_v, t6_v, degacc_sh):
  core = lax.axis_index("c")
  sid = lax.axis_index("s")
  ebase = sid * EPT
  nioff = core * E_PAD
  tslice = sid * RPT
  coreN = core * N

  fz16 = jnp.zeros((16,), jnp.float32)
  iota16 = lax.iota(jnp.int32, 16)
  z16i = jnp.zeros((16,), jnp.int32)

  @pl.loop(0, B)
  def _(r):
    for c in range(F // 16):
      msg_v[r, pl.ds(c * 16, 16)] = fz16

  off = 0
  for ln in (B,) * 9 + (48,):
    pltpu.sync_copy(msg_v.at[pl.ds(0, ln)],
                    degacc_sh.at[pl.ds(tslice + off, ln)])
    off += ln

  @pl.when(sid == NS - 1)
  def _():
    pltpu.sync_copy(msg_v.at[pl.ds(0, REM)],
                    degacc_sh.at[pl.ds(NS * RPT, REM)])

  plsc.subcore_barrier()

  @pl.loop(0, EPT, step=B)
  def _(eoff):
    src = ebase + eoff
    pltpu.sync_copy(niA_hbm.at[pl.ds(nioff + src, B)], niB_v)
    pltpu.sync_copy(w_hbm.at[pl.ds(src, B)], w_v)
    for e0 in range(0, B, 16):
      w16 = w_v[pl.ds(e0, 16)]
      plsc.store_scatter(msg_v, [iota16 + e0, z16i], w16)
    pltpu.sync_copy(msg_v, degacc_sh.at[niB_v], add=True)

  plsc.subcore_barrier()

  for c0 in range(0, RPT - 48, B):
    pltpu.sync_copy(degacc_sh.at[pl.ds(tslice + c0, B)], msg_v)
    for rr in range(0, B, 16):
      t6_v[pl.ds(c0 + rr, 16)] = plsc.load_gather(
          msg_v, [iota16 + rr, z16i])
  pltpu.sync_copy(degacc_sh.at[pl.ds(tslice + RPT - 48, 48)],
                  msg_v.at[pl.ds(0, 48)])
  for rr in range(0, 48, 16):
    t6_v[pl.ds(RPT - 48 + rr, 16)] = plsc.load_gather(
        msg_v, [iota16 + rr, z16i])
  pltpu.sync_copy(t6_v, deg_hbm.at[pl.ds(coreN + tslice, RPT)])

  @pl.when(sid == NS - 1)
  def _():
    pltpu.sync_copy(degacc_sh.at[pl.ds(NS * RPT, REM)],
                    msg_v.at[pl.ds(0, REM)])
    t6_v[pl.ds(0, 16)] = plsc.load_gather(msg_v, [iota16, z16i])
    pltpu.sync_copy(t6_v.at[pl.ds(0, REM)],
                    deg_hbm.at[pl.ds(coreN + NS * RPT, REM)])


def _sc_degrees(niA, w):
  mesh = plsc.VectorSubcoreMesh(core_axis_name="c", subcore_axis_name="s")
  scratch = [
      pltpu.VMEM((B, F), jnp.float32),
      pltpu.VMEM((B,), jnp.int32),
      pltpu.VMEM((B,), jnp.float32),
      pltpu.VMEM((RPT,), jnp.float32),
      pltpu.VMEM_SHARED((N, F), jnp.float32),
  ]
  f = pl.kernel(_deg_body,
                out_type=jax.ShapeDtypeStruct((2 * N,), jnp.float32),
                mesh=mesh, scratch_types=scratch,
                compiler_params=_sc_compiler_params())
  return f(niA, w)


def _prop_body(x_hbm, ri_hbm, sc_hbm, niA_hbm, rig_hbm, w_hbm, deg_hbm,
               p1_hbm, s2_hbm,
               rows_v, ric_v, ni_v, w_v, wn_v, sc_v, deg1d_v, acc_sh):
  core = lax.axis_index("c")
  sid = lax.axis_index("s")
  ebase = sid * EPT
  nioff = core * E_PAD
  tslice = sid * RPT
  coreN = core * N

  fz16 = jnp.zeros((16,), jnp.float32)
  z16i = jnp.zeros((16,), jnp.int32)

  pltpu.sync_copy(deg_hbm.at[pl.ds(coreN, N)], deg1d_v)

  def zero_rows_v():
    @pl.loop(0, B)
    def _(r):
      for c in range(F // 16):
        rows_v[r, pl.ds(c * 16, 16)] = fz16

  def zero_acc_slice():
    off = 0
    for ln in (B,) * 9 + (48,):
      pltpu.sync_copy(rows_v.at[pl.ds(0, ln)],
                      acc_sh.at[pl.ds(tslice + off, ln)])
      off += ln

    @pl.when(sid == NS - 1)
    def _():
      pltpu.sync_copy(rows_v.at[pl.ds(0, REM)],
                      acc_sh.at[pl.ds(NS * RPT, REM)])

  def copy_acc_out(dst_hbm):
    off = 0
    for ln in (B,) * 9 + (48,):
      pltpu.sync_copy(acc_sh.at[pl.ds(tslice + off, ln)],
                      rows_v.at[pl.ds(0, ln)])
      pltpu.sync_copy(rows_v.at[pl.ds(0, ln)],
                      dst_hbm.at[pl.ds(coreN + tslice + off, ln)])
      off += ln

    @pl.when(sid == NS - 1)
    def _():
      pltpu.sync_copy(acc_sh.at[pl.ds(NS * RPT, REM)],
                      rows_v.at[pl.ds(0, REM)])
      pltpu.sync_copy(rows_v.at[pl.ds(0, REM)],
                      dst_hbm.at[pl.ds(coreN + NS * RPT, REM)])

  zero_rows_v()
  zero_acc_slice()
  plsc.subcore_barrier()

  def run_round(table_hbm, gidx_hbm, gidx_off):
    @pl.loop(0, EPT, step=MEGA)
    def _(moff):
      src = ebase + moff
      pltpu.sync_copy(niA_hbm.at[pl.ds(nioff + src, MEGA)], ni_v)
      pltpu.sync_copy(w_hbm.at[pl.ds(src, MEGA)], w_v)

      @pl.loop(0, MEGA, step=B)
      def _(cb):
        pltpu.sync_copy(sc_hbm.at[pl.ds(src + cb, B)], sc_v)
        pltpu.sync_copy(gidx_hbm.at[pl.ds(gidx_off + src + cb, B)], ric_v)
        for e0 in range(0, B, 16):
          ni16 = ni_v[pl.ds(cb + e0, 16)]
          dv = plsc.load_gather(deg1d_v, [ni16])
          w16 = w_v[pl.ds(cb + e0, 16)]
          wn_v[pl.ds(cb + e0, 16)] = w16 / jnp.maximum(dv, EPS)
        pltpu.sync_copy(table_hbm.at[ric_v], rows_v)

        @pl.loop(0, B)
        def _(e):
          wf = plsc.load_gather(wn_v, [z16i + (cb + e)])
          for c in range(F // 16):
            sl = pl.ds(c * 16, 16)
            rows_v[e, sl] = rows_v[e, sl] * wf

        pltpu.sync_copy(rows_v, acc_sh.at[sc_v], add=True)

  run_round(x_hbm, ri_hbm, 0)
  plsc.subcore_barrier()
  copy_acc_out(p1_hbm)
  zero_rows_v()
  zero_acc_slice()
  plsc.subcore_barrier()

  run_round(p1_hbm, rig_hbm, nioff)
  plsc.subcore_barrier()
  copy_acc_out(s2_hbm)


def _sc_prop(x, ri, sc_, niA, rig, w, deg):
  mesh = plsc.VectorSubcoreMesh(core_axis_name="c", subcore_axis_name="s")
  out_types = (jax.ShapeDtypeStruct((2 * N, F), jnp.float32),
               jax.ShapeDtypeStruct((2 * N, F), jnp.float32))
  scratch = [
      pltpu.VMEM((B, F), jnp.float32),
      pltpu.VMEM((B,), jnp.int32),
      pltpu.VMEM((MEGA,), jnp.int32),
      pltpu.VMEM((MEGA,), jnp.float32),
      pltpu.VMEM((MEGA,), jnp.float32),
      pltpu.VMEM((B,), jnp.int32),
      pltpu.VMEM((N,), jnp.float32),
      pltpu.VMEM_SHARED((N, F), jnp.float32),
  ]
  f = pl.kernel(_prop_body, out_type=out_types, mesh=mesh,
                scratch_types=scratch,
                compiler_params=_sc_compiler_params())
  return f(x, ri, sc_, niA, rig, w, deg)


def _tc_body(x_r, p1i_r, p1o_r, s2i_r, s2o_r, wc_r, bc_r, wl_r, bl_r, o_r):
  wc = wc_r[...]
  hp = jax.lax.Precision.HIGHEST
  G = (jnp.dot(x_r[...], wc[0:128], preferred_element_type=jnp.float32,
               precision=hp)
       + jnp.dot(p1i_r[...], wc[128:256], preferred_element_type=jnp.float32,
                 precision=hp)
       + jnp.dot(p1o_r[...], wc[256:384], preferred_element_type=jnp.float32,
                 precision=hp)
       + jnp.dot(s2i_r[...], wc[384:512], preferred_element_type=jnp.float32,
                 precision=hp)
       + jnp.dot(s2o_r[...], wc[512:640], preferred_element_type=jnp.float32,
                 precision=hp)
       + bc_r[...])
  Z = jax.nn.sigmoid(G[:, :F])
  T = jnp.tanh(G[:, F:])
  H = (1.0 - Z) * T
  alpha = 1.6732632423543772
  scale = 1.0507009873554805
  h = scale * jnp.where(H > 0, H, alpha * (jnp.exp(H) - 1.0))
  o_r[...] = jnp.dot(h, wl_r[...], preferred_element_type=jnp.float32,
                     precision=hp) + bl_r[...]


def _tc_final(x, p1, s2, Wcat, bcat, Wl, bl):
  BLK = 400
  nblk = N // BLK
  out_dim = Wl.shape[1]
  return pl.pallas_call(
      _tc_body,
      grid=(nblk,),
      in_specs=[
          pl.BlockSpec((BLK, F), lambda i: (i, 0)),
          pl.BlockSpec((BLK, F), lambda i: (i, 0)),
          pl.BlockSpec((BLK, F), lambda i, _n=nblk: (i + _n, 0)),
          pl.BlockSpec((BLK, F), lambda i: (i, 0)),
          pl.BlockSpec((BLK, F), lambda i, _n=nblk: (i + _n, 0)),
          pl.BlockSpec((5 * F, 2 * F), lambda i: (0, 0)),
          pl.BlockSpec((1, 2 * F), lambda i: (0, 0)),
          pl.BlockSpec((F, out_dim), lambda i: (0, 0)),
          pl.BlockSpec((1, out_dim), lambda i: (0, 0)),
      ],
      out_specs=pl.BlockSpec((BLK, out_dim), lambda i: (i, 0)),
      out_shape=jax.ShapeDtypeStruct((N, out_dim), jnp.float32),
  )(x, p1, p1, s2, s2, Wcat, bcat, Wl, bl)


def _cat_w(W):
  A = W[0, 0, :F] + W[1, 0, :F] - W[0, 2, :F] - W[1, 2, :F]
  return jnp.concatenate(
      [A, W[1, 1, :F], W[0, 1, :F], 2.0 * W[1, 2, :F], 2.0 * W[0, 2, :F]],
      axis=0)


def kernel(x, edge_index, edge_weight, Wz, bz, Wr, br, Wh, bh, Wl, bl):
  row = edge_index[0]
  col = edge_index[1]
  pad = E_PAD - E
  ri = jnp.pad(row, (0, pad))
  sc_ = jnp.pad(col, (0, pad))
  w = jnp.pad(edge_weight, (0, pad))
  niA = jnp.concatenate([sc_, ri])
  rig = jnp.concatenate([ri, ri + N])

  deg = _sc_degrees(niA, w)
  p1, s2 = _sc_prop(x, ri, sc_, niA, rig, w, deg)

  Wcat = jnp.concatenate([_cat_w(Wz), _cat_w(Wh)], axis=1)
  bcat = jnp.concatenate([bz, bh]).reshape(1, 2 * F)
  out = _tc_final(x, p1, s2, Wcat, bcat, Wl, bl.reshape(1, -1))
  return out


# speedup vs baseline: 7.2198x; 1.1084x over previous
import dataclasses
import functools

import jax
import jax.numpy as jnp
from jax import lax
from jax.experimental import pallas as pl
from jax.experimental.pallas import tpu as pltpu
from jax.experimental.pallas import tpu_sc as plsc

N = 10000
E = 320000
F = 128
NS = 16
B = 64
MEGA = 512
EPT = 20480
E_PAD = EPT * NS
RPT = 624
REM = N - NS * RPT
EPS = 1e-30


def _sc_compiler_params():
  cp = pltpu.CompilerParams()
  if "needs_layout_passes" in pltpu.CompilerParams.__dataclass_fields__:
    cp = dataclasses.replace(cp, needs_layout_passes=False)
  return cp


def _deg_body(niA_hbm, w_hbm, deg_hbm,
              msg_v, niB_v, w_v, t6_v, degacc_sh):
  core = lax.axis_index("c")
  sid = lax.axis_index("s")
  ebase = sid * EPT
  nioff = core * E_PAD
  tslice = sid * RPT
  coreN = core * N

  fz16 = jnp.zeros((16,), jnp.float32)
  iota16 = lax.iota(jnp.int32, 16)
  z16i = jnp.zeros((16,), jnp.int32)

  @pl.loop(0, B)
  def _(r):
    for c in range(F // 16):
      msg_v[r, pl.ds(c * 16, 16)] = fz16

  off = 0
  for ln in (B,) * 9 + (48,):
    pltpu.sync_copy(msg_v.at[pl.ds(0, ln)],
                    degacc_sh.at[pl.ds(tslice + off, ln)])
    off += ln

  @pl.when(sid == NS - 1)
  def _():
    pltpu.sync_copy(msg_v.at[pl.ds(0, REM)],
                    degacc_sh.at[pl.ds(NS * RPT, REM)])

  plsc.subcore_barrier()

  @pl.loop(0, EPT, step=B)
  def _(eoff):
    src = ebase + eoff
    pltpu.sync_copy(niA_hbm.at[pl.ds(nioff + src, B)], niB_v)
    pltpu.sync_copy(w_hbm.at[pl.ds(src, B)], w_v)
    for e0 in range(0, B, 16):
      w16 = w_v[pl.ds(e0, 16)]
      plsc.store_scatter(msg_v, [iota16 + e0, z16i], w16)
    pltpu.sync_copy(msg_v, degacc_sh.at[niB_v], add=True)

  plsc.subcore_barrier()

  for c0 in range(0, RPT - 48, B):
    pltpu.sync_copy(degacc_sh.at[pl.ds(tslice + c0, B)], msg_v)
    for rr in range(0, B, 16):
      t6_v[pl.ds(c0 + rr, 16)] = plsc.load_gather(
          msg_v, [iota16 + rr, z16i])
  pltpu.sync_copy(degacc_sh.at[pl.ds(tslice + RPT - 48, 48)],
                  msg_v.at[pl.ds(0, 48)])
  for rr in range(0, 48, 16):
    t6_v[pl.ds(RPT - 48 + rr, 16)] = plsc.load_gather(
        msg_v, [iota16 + rr, z16i])
  pltpu.sync_copy(t6_v, deg_hbm.at[pl.ds(coreN + tslice, RPT)])

  @pl.when(sid == NS - 1)
  def _():
    pltpu.sync_copy(degacc_sh.at[pl.ds(NS * RPT, REM)],
                    msg_v.at[pl.ds(0, REM)])
    t6_v[pl.ds(0, 16)] = plsc.load_gather(msg_v, [iota16, z16i])
    pltpu.sync_copy(t6_v.at[pl.ds(0, REM)],
                    deg_hbm.at[pl.ds(coreN + NS * RPT, REM)])


def _sc_degrees(niA, w):
  mesh = plsc.VectorSubcoreMesh(core_axis_name="c", subcore_axis_name="s")
  scratch = [
      pltpu.VMEM((B, F), jnp.float32),
      pltpu.VMEM((B,), jnp.int32),
      pltpu.VMEM((B,), jnp.float32),
      pltpu.VMEM((RPT,), jnp.float32),
      pltpu.VMEM_SHARED((N, F), jnp.float32),
  ]
  f = pl.kernel(_deg_body,
                out_type=jax.ShapeDtypeStruct((2 * N,), jnp.float32),
                mesh=mesh, scratch_types=scratch,
                compiler_params=_sc_compiler_params())
  return f(niA, w)


def _prop_body(x_hbm, ri_hbm, sc_hbm, niA_hbm, rig_hbm, w_hbm, deg_hbm,
               p1_hbm, s2_hbm,
               rows_v, rows2_v, ric_v, ric2_v, ni_v, w_v, wn_v, sc_v, sc2_v,
               deg1d_v, gsem0, gsem1, ssem0, ssem1, acc_sh):
  core = lax.axis_index("c")
  sid = lax.axis_index("s")
  ebase = sid * EPT
  nioff = core * E_PAD
  tslice = sid * RPT
  coreN = core * N

  fz16 = jnp.zeros((16,), jnp.float32)
  z16i = jnp.zeros((16,), jnp.int32)

  pltpu.sync_copy(deg_hbm.at[pl.ds(coreN, N)], deg1d_v)

  def zero_rows_v():
    @pl.loop(0, B)
    def _(r):
      for c in range(F // 16):
        rows_v[r, pl.ds(c * 16, 16)] = fz16

  def zero_acc_slice():
    off = 0
    for ln in (B,) * 9 + (48,):
      pltpu.sync_copy(rows_v.at[pl.ds(0, ln)],
                      acc_sh.at[pl.ds(tslice + off, ln)])
      off += ln

    @pl.when(sid == NS - 1)
    def _():
      pltpu.sync_copy(rows_v.at[pl.ds(0, REM)],
                      acc_sh.at[pl.ds(NS * RPT, REM)])

  def copy_acc_out(dst_hbm):
    off = 0
    for ln in (B,) * 9 + (48,):
      pltpu.sync_copy(acc_sh.at[pl.ds(tslice + off, ln)],
                      rows_v.at[pl.ds(0, ln)])
      pltpu.sync_copy(rows_v.at[pl.ds(0, ln)],
                      dst_hbm.at[pl.ds(coreN + tslice + off, ln)])
      off += ln

    @pl.when(sid == NS - 1)
    def _():
      pltpu.sync_copy(acc_sh.at[pl.ds(NS * RPT, REM)],
                      rows_v.at[pl.ds(0, REM)])
      pltpu.sync_copy(rows_v.at[pl.ds(0, REM)],
                      dst_hbm.at[pl.ds(coreN + NS * RPT, REM)])

  zero_rows_v()
  zero_acc_slice()
  plsc.subcore_barrier()

  def run_round(table_hbm, gidx_hbm, gidx_off):
    @pl.loop(0, EPT, step=MEGA)
    def _(moff):
      src = ebase + moff
      pltpu.sync_copy(niA_hbm.at[pl.ds(nioff + src, MEGA)], ni_v)
      pltpu.sync_copy(w_hbm.at[pl.ds(src, MEGA)], w_v)

      @pl.loop(0, MEGA, step=2 * B)
      def _(cb0):
        pltpu.sync_copy(sc_hbm.at[pl.ds(src + cb0, B)], sc_v)
        pltpu.sync_copy(sc_hbm.at[pl.ds(src + cb0 + B, B)], sc2_v)
        pltpu.sync_copy(gidx_hbm.at[pl.ds(gidx_off + src + cb0, B)], ric_v)
        pltpu.sync_copy(gidx_hbm.at[pl.ds(gidx_off + src + cb0 + B, B)],
                        ric2_v)
        g0 = pltpu.async_copy(table_hbm.at[ric_v], rows_v, gsem0)
        g1 = pltpu.async_copy(table_hbm.at[ric2_v], rows2_v, gsem1)
        scats = []
        for cb, rows, scref, g, ssem in (
            (cb0, rows_v, sc_v, g0, ssem0),
            (cb0 + B, rows2_v, sc2_v, g1, ssem1)):
          for e0 in range(0, B, 16):
            ni16 = ni_v[pl.ds(cb + e0, 16)]
            dv = plsc.load_gather(deg1d_v, [ni16])
            w16 = w_v[pl.ds(cb + e0, 16)]
            wn_v[pl.ds(cb + e0, 16)] = w16 / jnp.maximum(dv, EPS)
          g.wait()

          @pl.loop(0, B)
          def _(e, _rows=rows, _cb=cb):
            wf = plsc.load_gather(wn_v, [z16i + (_cb + e)])
            for c in range(F // 16):
              sl = pl.ds(c * 16, 16)
              _rows[e, sl] = _rows[e, sl] * wf

          scats.append(
              pltpu.async_copy(rows, acc_sh.at[scref], ssem, add=True))
        for s_ in scats:
          s_.wait()

  run_round(x_hbm, ri_hbm, 0)
  plsc.subcore_barrier()
  copy_acc_out(p1_hbm)
  zero_rows_v()
  zero_acc_slice()
  plsc.subcore_barrier()

  run_round(p1_hbm, rig_hbm, nioff)
  plsc.subcore_barrier()
  copy_acc_out(s2_hbm)


def _sc_prop(x, ri, sc_, niA, rig, w, deg):
  mesh = plsc.VectorSubcoreMesh(core_axis_name="c", subcore_axis_name="s")
  out_types = (jax.ShapeDtypeStruct((2 * N, F), jnp.float32),
               jax.ShapeDtypeStruct((2 * N, F), jnp.float32))
  scratch = [
      pltpu.VMEM((B, F), jnp.float32),
      pltpu.VMEM((B, F), jnp.float32),
      pltpu.VMEM((B,), jnp.int32),
      pltpu.VMEM((B,), jnp.int32),
      pltpu.VMEM((MEGA,), jnp.int32),
      pltpu.VMEM((MEGA,), jnp.float32),
      pltpu.VMEM((MEGA,), jnp.float32),
      pltpu.VMEM((B,), jnp.int32),
      pltpu.VMEM((B,), jnp.int32),
      pltpu.VMEM((N,), jnp.float32),
      pltpu.SemaphoreType.DMA,
      pltpu.SemaphoreType.DMA,
      pltpu.SemaphoreType.DMA,
      pltpu.SemaphoreType.DMA,
      pltpu.VMEM_SHARED((N, F), jnp.float32),
  ]
  f = pl.kernel(_prop_body, out_type=out_types, mesh=mesh,
                scratch_types=scratch,
                compiler_params=_sc_compiler_params())
  return f(x, ri, sc_, niA, rig, w, deg)


def _tc_body(x_r, p1i_r, p1o_r, s2i_r, s2o_r, wc_r, bc_r, wl_r, bl_r, o_r):
  wc = wc_r[...]
  hp = jax.lax.Precision.HIGHEST
  G = (jnp.dot(x_r[...], wc[0:128], preferred_element_type=jnp.float32,
               precision=hp)
       + jnp.dot(p1i_r[...], wc[128:256], preferred_element_type=jnp.float32,
                 precision=hp)
       + jnp.dot(p1o_r[...], wc[256:384], preferred_element_type=jnp.float32,
                 precision=hp)
       + jnp.dot(s2i_r[...], wc[384:512], preferred_element_type=jnp.float32,
                 precision=hp)
       + jnp.dot(s2o_r[...], wc[512:640], preferred_element_type=jnp.float32,
                 precision=hp)
       + bc_r[...])
  Z = jax.nn.sigmoid(G[:, :F])
  T = jnp.tanh(G[:, F:])
  H = (1.0 - Z) * T
  alpha = 1.6732632423543772
  scale = 1.0507009873554805
  h = scale * jnp.where(H > 0, H, alpha * (jnp.exp(H) - 1.0))
  o_r[...] = jnp.dot(h, wl_r[...], preferred_element_type=jnp.float32,
                     precision=hp) + bl_r[...]


def _tc_final(x, p1, s2, Wcat, bcat, Wl, bl):
  BLK = 400
  nblk = N // BLK
  out_dim = Wl.shape[1]
  return pl.pallas_call(
      _tc_body,
      grid=(nblk,),
      in_specs=[
          pl.BlockSpec((BLK, F), lambda i: (i, 0)),
          pl.BlockSpec((BLK, F), lambda i: (i, 0)),
          pl.BlockSpec((BLK, F), lambda i, _n=nblk: (i + _n, 0)),
          pl.BlockSpec((BLK, F), lambda i: (i, 0)),
          pl.BlockSpec((BLK, F), lambda i, _n=nblk: (i + _n, 0)),
          pl.BlockSpec((5 * F, 2 * F), lambda i: (0, 0)),
          pl.BlockSpec((1, 2 * F), lambda i: (0, 0)),
          pl.BlockSpec((F, out_dim), lambda i: (0, 0)),
          pl.BlockSpec((1, out_dim), lambda i: (0, 0)),
      ],
      out_specs=pl.BlockSpec((BLK, out_dim), lambda i: (i, 0)),
      out_shape=jax.ShapeDtypeStruct((N, out_dim), jnp.float32),
  )(x, p1, p1, s2, s2, Wcat, bcat, Wl, bl)


def _cat_w(W):
  A = W[0, 0, :F] + W[1, 0, :F] - W[0, 2, :F] - W[1, 2, :F]
  return jnp.concatenate(
      [A, W[1, 1, :F], W[0, 1, :F], 2.0 * W[1, 2, :F], 2.0 * W[0, 2, :F]],
      axis=0)


def kernel(x, edge_index, edge_weight, Wz, bz, Wr, br, Wh, bh, Wl, bl):
  row = edge_index[0]
  col = edge_index[1]
  pad = E_PAD - E
  ri = jnp.pad(row, (0, pad))
  sc_ = jnp.pad(col, (0, pad))
  w = jnp.pad(edge_weight, (0, pad))
  niA = jnp.concatenate([sc_, ri])
  rig = jnp.concatenate([ri, ri + N])

  deg = _sc_degrees(niA, w)
  p1, s2 = _sc_prop(x, ri, sc_, niA, rig, w, deg)

  Wcat = jnp.concatenate([_cat_w(Wz), _cat_w(Wh)], axis=1)
  bcat = jnp.concatenate([bz, bh]).reshape(1, 2 * F)
  out = _tc_final(x, p1, s2, Wcat, bcat, Wl, bl.reshape(1, -1))
  return out
